# Initial kernel scaffold; baseline (speedup 1.0000x reference)
import functools

import jax
import jax.numpy as jnp
from jax import lax
from jax.experimental import pallas as pl
from jax.experimental.pallas import tpu as pltpu
from jax.experimental.pallas import tpu_sc as plsc

_N = 10000
_E = 320000
_DIN = 128
_HID = 256
_HEADS = 4
_B = 64
_L = 3

_NP = 10240
_NW = 32
_NPT = _NP // _NW
_ETOT = _E + _N
_EPAD = 330064
_NEG = -3.0e38


def _exti(ref, pos):
    blk = jnp.bitwise_and(pos, jnp.int32(-16))
    v = ref[pl.ds(blk, 16)]
    lane = lax.iota(jnp.int32, 16) + blk
    return jnp.max(jnp.where(lane == pos, v, jnp.int32(-2147483647)))


def _full16(val):
    return jnp.full((16,), val, jnp.int32)


def _gat_sc_body(srcs_h, rp_h, tbl_h, xp_h, out_h,
                 tbl_v, rp_v, sidx_v, rows_v, acc_v, ebuf_v, sem):
    c = lax.axis_index("c")
    s = lax.axis_index("s")
    wid = s * 2 + c
    n0 = wid * _NPT
    pltpu.sync_copy(tbl_h, tbl_v)
    pltpu.sync_copy(rp_h.at[pl.ds(n0, _NPT + 16)], rp_v)
    lanes = lax.iota(jnp.int32, 16)

    def node_body(nl, _):
        n = n0 + nl
        e0 = _exti(rp_v, nl)
        e1 = _exti(rp_v, nl + 1)
        a0 = jnp.bitwise_and(e0, jnp.int32(-16))
        nblk = (e1 - a0 + 15) >> 4
        nvec = _full16(n)

        def alpha_heads(eb):
            pltpu.sync_copy(srcs_h.at[pl.ds(eb, 16)], sidx_v)
            sidx = sidx_v[...]
            valid = (lanes + eb >= e0) & (lanes + eb < e1)
            als = []
            for h in range(_HEADS):
                asv = plsc.load_gather(tbl_v, [_full16(h), sidx])
                adv = plsc.load_gather(tbl_v, [_full16(4 + h), nvec])
                al = asv + adv
                al = jnp.where(al >= 0.0, al, al * jnp.float32(0.2))
                als.append(al)
            return valid, als

        def blk_a(b, ms):
            eb = a0 + b * 16
            valid, als = alpha_heads(eb)
            return tuple(jnp.maximum(ms[h], jnp.where(valid, als[h], _NEG))
                         for h in range(_HEADS))

        ms = lax.fori_loop(0, nblk, blk_a,
                           tuple(jnp.full((16,), _NEG, jnp.float32)
                                 for _ in range(_HEADS)))
        amax = [jnp.max(ms[h]) for h in range(_HEADS)]

        for t in range(64):
            acc_v[pl.ds(t * 16, 16)] = jnp.zeros((16,), jnp.float32)

        def blk_b(b, dsum):
            eb = a0 + b * 16
            valid, als = alpha_heads(eb)
            cp = pltpu.async_copy(xp_h.at[sidx_v], rows_v, sem)
            new_dsum = []
            for h in range(_HEADS):
                ev = jnp.exp(als[h] - amax[h])
                ev = jnp.where(valid, ev, jnp.float32(0.0))
                ebuf_v[h, :] = ev
                new_dsum.append(dsum[h] + ev)
            cp.wait()

            def row_body(r, _):
                row = rows_v.at[r]
                for h in range(_HEADS):
                    evr = plsc.load_gather(ebuf_v, [_full16(h), _full16(r)])
                    for t in range(16):
                        off = h * 256 + t * 16
                        acc_v[pl.ds(off, 16)] = (acc_v[pl.ds(off, 16)]
                                                 + row[pl.ds(off, 16)] * evr)
                return 0

            lax.fori_loop(0, 16, row_body, 0)
            return tuple(new_dsum)

        dsum = lax.fori_loop(0, nblk, blk_b,
                             tuple(jnp.zeros((16,), jnp.float32)
                                   for _ in range(_HEADS)))
        for h in range(_HEADS):
            inv = jnp.float32(1.0) / (jnp.sum(dsum[h]) + jnp.float32(1e-16))
            for t in range(16):
                off = h * 256 + t * 16
                acc_v[pl.ds(off, 16)] = acc_v[pl.ds(off, 16)] * inv
        pltpu.sync_copy(acc_v, out_h.at[n])
        return 0

    lax.fori_loop(0, _NPT, node_body, 0)


def _gat_aggregate(srcs_pad, rp_pad, tbl, xp):
    mesh = plsc.VectorSubcoreMesh(core_axis_name="c", subcore_axis_name="s")
    return pl.kernel(
        _gat_sc_body,
        out_type=jax.ShapeDtypeStruct((_NP, _HEADS * _HID), jnp.float32),
        mesh=mesh,
        scratch_types=[
            pltpu.VMEM((8, _NP), jnp.float32),
            pltpu.VMEM((_NPT + 16,), jnp.int32),
            pltpu.VMEM((16,), jnp.int32),
            pltpu.VMEM((16, _HEADS * _HID), jnp.float32),
            pltpu.VMEM((_HEADS * _HID,), jnp.float32),
            pltpu.VMEM((_HEADS, 16), jnp.float32),
            pltpu.SemaphoreType.DMA,
        ],
    )(srcs_pad, rp_pad, tbl, xp)


def _in_proj_body(x_ref, w_ref, b_ref, o_ref):
    o_ref[...] = jnp.maximum(
        jnp.dot(x_ref[...], w_ref[...], preferred_element_type=jnp.float32)
        + b_ref[...], 0.0)


def _xp_body(h_ref, w_ref, as_ref, ad_ref, xp_ref, tb_ref):
    xp = jnp.dot(h_ref[...], w_ref[...], preferred_element_type=jnp.float32)
    xp_ref[...] = xp
    rows = []
    for h in range(_HEADS):
        xh = xp[:, h * _HID:(h + 1) * _HID]
        rows.append(jnp.sum(xh * as_ref[h, :][None, :], axis=1)[None, :])
    for h in range(_HEADS):
        xh = xp[:, h * _HID:(h + 1) * _HID]
        rows.append(jnp.sum(xh * ad_ref[h, :][None, :], axis=1)[None, :])
    tb_ref[...] = jnp.concatenate(rows, axis=0)


def _upd_body(os_ref, h_ref, bg_ref, o_ref):
    i = pl.program_id(0)
    sblk = os_ref[...]
    hm = (sblk[:, 0:256] + sblk[:, 256:512]
          + sblk[:, 512:768] + sblk[:, 768:1024]) * 0.25
    hn = jnp.maximum(hm + bg_ref[...], 0.0)
    rows = i * sblk.shape[0] + lax.broadcasted_iota(
        jnp.int32, (sblk.shape[0], _HID), 0)
    o_ref[...] = h_ref[...] + jnp.where(rows < _N, hn, 0.0)


def _pool_body(h_ref, b_ref, wa1_ref, ba1_ref, wa2_ref, ba2_ref, o_ref,
               acc, cnt):
    j = pl.program_id(0)

    @pl.when(j == 0)
    def _():
        acc[...] = jnp.zeros_like(acc)
        cnt[...] = jnp.zeros_like(cnt)

    bn = h_ref.shape[0]
    P = (b_ref[...] == lax.broadcasted_iota(jnp.int32, (_B, bn), 0)
         ).astype(jnp.float32)
    acc[...] += jnp.dot(P, h_ref[...], preferred_element_type=jnp.float32)
    cnt[...] += jnp.sum(P, axis=1, keepdims=True)

    @pl.when(j == pl.num_programs(0) - 1)
    def _():
        pooled = acc[...] / jnp.maximum(cnt[...][:, 0:1], 1.0)
        t = jnp.maximum(
            jnp.dot(pooled, wa1_ref[...], preferred_element_type=jnp.float32)
            + ba1_ref[...], 0.0)
        o_ref[...] = (jnp.dot(t, wa2_ref[...],
                              preferred_element_type=jnp.float32)
                      + ba2_ref[...])


def kernel(x, edge_index, batch, W0, b0, Wg, att_src, att_dst, bg,
           Wa1, ba1, Wa2, ba2):
    loops = jnp.arange(_N, dtype=edge_index.dtype)
    src = jnp.concatenate([edge_index[0], loops])
    dst = jnp.concatenate([edge_index[1], loops])
    order = jnp.argsort(dst)
    srcs = src[order].astype(jnp.int32)
    dsts = dst[order].astype(jnp.int32)
    srcs_pad = jnp.concatenate(
        [srcs, jnp.zeros((_EPAD - _ETOT,), jnp.int32)])
    rp_pad = jnp.searchsorted(
        dsts, jnp.arange(_NP + 16, dtype=jnp.int32), side="left"
    ).astype(jnp.int32)
    x_pad = jnp.pad(x, ((0, _NP - _N), (0, 0)))
    batch_pad = jnp.pad(batch.astype(jnp.int32), (0, _NP - _N),
                        constant_values=_B).reshape(1, _NP)
    b0_2d = b0.reshape(1, _HID)

    bn = 1024
    h = pl.pallas_call(
        _in_proj_body,
        grid=(_NP // bn,),
        in_specs=[
            pl.BlockSpec((bn, _DIN), lambda i: (i, 0)),
            pl.BlockSpec((_DIN, _HID), lambda i: (0, 0)),
            pl.BlockSpec((1, _HID), lambda i: (0, 0)),
        ],
        out_specs=pl.BlockSpec((bn, _HID), lambda i: (i, 0)),
        out_shape=jax.ShapeDtypeStruct((_NP, _HID), jnp.float32),
    )(x_pad, W0, b0_2d)

    bx = 512
    for i in range(_L):
        asp = jnp.pad(att_src[i], ((0, 8 - _HEADS), (0, 0)))
        adp = jnp.pad(att_dst[i], ((0, 8 - _HEADS), (0, 0)))
        xp, tbl = pl.pallas_call(
            _xp_body,
            grid=(_NP // bx,),
            in_specs=[
                pl.BlockSpec((bx, _HID), lambda j: (j, 0)),
                pl.BlockSpec((_HID, _HEADS * _HID), lambda j: (0, 0)),
                pl.BlockSpec((8, _HID), lambda j: (0, 0)),
                pl.BlockSpec((8, _HID), lambda j: (0, 0)),
            ],
            out_specs=[
                pl.BlockSpec((bx, _HEADS * _HID), lambda j: (j, 0)),
                pl.BlockSpec((8, bx), lambda j: (0, j)),
            ],
            out_shape=[
                jax.ShapeDtypeStruct((_NP, _HEADS * _HID), jnp.float32),
                jax.ShapeDtypeStruct((8, _NP), jnp.float32),
            ],
        )(h, Wg[i], asp, adp)

        out_sum = _gat_aggregate(srcs_pad, rp_pad, tbl, xp)

        h = pl.pallas_call(
            _upd_body,
            grid=(_NP // bx,),
            in_specs=[
                pl.BlockSpec((bx, _HEADS * _HID), lambda j: (j, 0)),
                pl.BlockSpec((bx, _HID), lambda j: (j, 0)),
                pl.BlockSpec((1, _HID), lambda j: (0, 0)),
            ],
            out_specs=pl.BlockSpec((bx, _HID), lambda j: (j, 0)),
            out_shape=jax.ShapeDtypeStruct((_NP, _HID), jnp.float32),
        )(out_sum, h, bg[i].reshape(1, _HID))

    bp = 2048
    out = pl.pallas_call(
        _pool_body,
        grid=(_NP // bp,),
        in_specs=[
            pl.BlockSpec((bp, _HID), lambda j: (j, 0)),
            pl.BlockSpec((1, bp), lambda j: (0, j)),
            pl.BlockSpec((_HID, 2 * _HID), lambda j: (0, 0)),
            pl.BlockSpec((1, 2 * _HID), lambda j: (0, 0)),
            pl.BlockSpec((2 * _HID, _HID), lambda j: (0, 0)),
            pl.BlockSpec((1, _HID), lambda j: (0, 0)),
        ],
        out_specs=pl.BlockSpec((_B, _HID), lambda j: (0, 0)),
        out_shape=jax.ShapeDtypeStruct((_B, _HID), jnp.float32),
        scratch_shapes=[
            pltpu.VMEM((_B, _HID), jnp.float32),
            pltpu.VMEM((_B, 1), jnp.float32),
        ],
    )(h, batch_pad, Wa1, ba1.reshape(1, 2 * _HID), Wa2,
      ba2.reshape(1, _HID))
    return out


# speedup vs baseline: 5.3706x; 5.3706x over previous
import functools

import jax
import jax.numpy as jnp
from jax import lax
from jax.experimental import pallas as pl
from jax.experimental.pallas import tpu as pltpu
from jax.experimental.pallas import tpu_sc as plsc

_N = 10000
_E = 320000
_DIN = 128
_HID = 256
_HEADS = 4
_B = 64
_L = 3

_NP = 10240
_NW = 32
_NPT = _NP // _NW
_ETOT = _E + _N
_EPAD = 330064
_NEG = -3.0e38


def _full16(val):
    return jnp.full((16,), val, jnp.int32)


def _exti(ref, pos):
    blk = pl.multiple_of(jnp.bitwise_and(pos, jnp.int32(-16)), 16)
    v = ref[pl.ds(blk, 16)]
    lane = lax.iota(jnp.int32, 16) + blk
    s = jnp.sum(jnp.where(lane == pos, v, jnp.float32(0.0)))
    return s.astype(jnp.int32)


def _gat_sc_body(srcs_h, rp_h, tbl_h, xp_h, out_h,
                 tbl_v, rp_v, sidx_v, rows_v, acc_v, ebuf_v, sem):
    c = lax.axis_index("c")
    s = lax.axis_index("s")
    wid = s * 2 + c
    n0 = pl.multiple_of(wid * _NPT, _NPT)
    pltpu.sync_copy(tbl_h, tbl_v)
    pltpu.sync_copy(rp_h.at[pl.ds(n0, _NPT + 16)], rp_v)
    lanes = lax.iota(jnp.int32, 16)

    def node_body(nl, _):
        n = n0 + nl
        e0 = _exti(rp_v, nl)
        e1 = _exti(rp_v, nl + 1)
        a0 = pl.multiple_of(jnp.bitwise_and(e0, jnp.int32(-16)), 16)
        nblk = (e1 - a0 + 15) >> 4
        nvec = _full16(n)

        def alpha_heads(eb):
            eb = pl.multiple_of(eb, 16)
            pltpu.sync_copy(srcs_h.at[pl.ds(eb, 16)], sidx_v)
            sidx = sidx_v[...]
            valid = (lanes + eb >= e0) & (lanes + eb < e1)
            als = []
            for h in range(_HEADS):
                asv = plsc.load_gather(tbl_v, [_full16(h * _NP) + sidx])
                adv = plsc.load_gather(tbl_v, [_full16((4 + h) * _NP) + nvec])
                al = asv + adv
                al = jnp.where(al >= 0.0, al, al * jnp.float32(0.2))
                als.append(al)
            return valid, als

        def blk_a(b, ms):
            eb = a0 + b * 16
            valid, als = alpha_heads(eb)
            return tuple(jnp.maximum(ms[h], jnp.where(valid, als[h], _NEG))
                         for h in range(_HEADS))

        ms = lax.fori_loop(0, nblk, blk_a,
                           tuple(jnp.full((16,), _NEG, jnp.float32)
                                 for _ in range(_HEADS)))
        amax = [jnp.max(ms[h]) for h in range(_HEADS)]

        for t in range(64):
            acc_v[pl.ds(t * 16, 16)] = jnp.zeros((16,), jnp.float32)

        def blk_b(b, dsum):
            eb = a0 + b * 16
            valid, als = alpha_heads(eb)
            cp = pltpu.async_copy(xp_h.at[sidx_v], rows_v, sem)
            new_dsum = []
            for h in range(_HEADS):
                ev = jnp.exp(als[h] - amax[h])
                ev = jnp.where(valid, ev, jnp.float32(0.0))
                ebuf_v[pl.ds(h * 16, 16)] = ev
                new_dsum.append(dsum[h] + ev)
            cp.wait()

            def row_body(r, _):
                rvec = _full16(r)
                for h in range(_HEADS):
                    evr = plsc.load_gather(ebuf_v, [_full16(h * 16) + rvec])
                    for t in range(16):
                        off = h * 256 + t * 16
                        rch = plsc.load_gather(rows_v, [rvec, _full16(off) + lanes])
                        acc_v[pl.ds(off, 16)] = (acc_v[pl.ds(off, 16)]
                                                 + rch * evr)
                return 0

            lax.fori_loop(0, 16, row_body, 0)
            return tuple(new_dsum)

        dsum = lax.fori_loop(0, nblk, blk_b,
                             tuple(jnp.zeros((16,), jnp.float32)
                                   for _ in range(_HEADS)))
        ones16 = jnp.ones((16,), jnp.float32)
        for h in range(_HEADS):
            dvec = jnp.full((16,), jnp.sum(dsum[h]), jnp.float32)
            inv = ones16 / (dvec + jnp.float32(1e-16))
            for t in range(16):
                off = h * 256 + t * 16
                acc_v[pl.ds(off, 16)] = acc_v[pl.ds(off, 16)] * inv
        pltpu.sync_copy(acc_v, out_h.at[n])
        return 0

    lax.fori_loop(0, _NPT, node_body, 0)


def _gat_aggregate(srcs_pad, rp_pad, tbl, xp):
    mesh = plsc.VectorSubcoreMesh(core_axis_name="c", subcore_axis_name="s",
                                  num_cores=2, num_subcores=16)
    return pl.kernel(
        _gat_sc_body,
        out_type=jax.ShapeDtypeStruct((_NP, _HEADS * _HID), jnp.float32),
        mesh=mesh,
        scratch_types=[
            pltpu.VMEM((8 * _NP,), jnp.float32),
            pltpu.VMEM((_NPT + 16,), jnp.float32),
            pltpu.VMEM((16,), jnp.int32),
            pltpu.VMEM((16, _HEADS * _HID), jnp.float32),
            pltpu.VMEM((_HEADS * _HID,), jnp.float32),
            pltpu.VMEM((_HEADS * 16,), jnp.float32),
            pltpu.SemaphoreType.DMA,
        ],
        compiler_params=pltpu.CompilerParams(needs_layout_passes=False),
    )(srcs_pad, rp_pad, tbl, xp)


def _in_proj_body(x_ref, w_ref, b_ref, o_ref):
    o_ref[...] = jnp.maximum(
        jnp.dot(x_ref[...], w_ref[...], preferred_element_type=jnp.float32)
        + b_ref[...], 0.0)


def _xp_body(h_ref, w_ref, as_ref, ad_ref, xp_ref, tb_ref):
    xp = jnp.dot(h_ref[...], w_ref[...], preferred_element_type=jnp.float32)
    xp_ref[...] = xp
    rows = []
    for h in range(_HEADS):
        xh = xp[:, h * _HID:(h + 1) * _HID]
        rows.append(jnp.sum(xh * as_ref[h, :][None, :], axis=1)[None, :])
    for h in range(_HEADS):
        xh = xp[:, h * _HID:(h + 1) * _HID]
        rows.append(jnp.sum(xh * ad_ref[h, :][None, :], axis=1)[None, :])
    tb_ref[...] = jnp.concatenate(rows, axis=0)


def _upd_body(os_ref, h_ref, bg_ref, o_ref):
    i = pl.program_id(0)
    sblk = os_ref[...]
    hm = (sblk[:, 0:256] + sblk[:, 256:512]
          + sblk[:, 512:768] + sblk[:, 768:1024]) * 0.25
    hn = jnp.maximum(hm + bg_ref[...], 0.0)
    rows = i * sblk.shape[0] + lax.broadcasted_iota(
        jnp.int32, (sblk.shape[0], _HID), 0)
    o_ref[...] = h_ref[...] + jnp.where(rows < _N, hn, 0.0)


def _pool_body(h_ref, b_ref, wa1_ref, ba1_ref, wa2_ref, ba2_ref, o_ref,
               acc, cnt):
    j = pl.program_id(0)

    @pl.when(j == 0)
    def _():
        acc[...] = jnp.zeros_like(acc)
        cnt[...] = jnp.zeros_like(cnt)

    bn = h_ref.shape[0]
    P = (b_ref[...] == lax.broadcasted_iota(jnp.int32, (_B, bn), 0)
         ).astype(jnp.float32)
    acc[...] += jnp.dot(P, h_ref[...], preferred_element_type=jnp.float32)
    cnt[...] += jnp.sum(P, axis=1, keepdims=True)

    @pl.when(j == pl.num_programs(0) - 1)
    def _():
        pooled = acc[...] / jnp.maximum(cnt[...][:, 0:1], 1.0)
        t = jnp.maximum(
            jnp.dot(pooled, wa1_ref[...], preferred_element_type=jnp.float32)
            + ba1_ref[...], 0.0)
        o_ref[...] = (jnp.dot(t, wa2_ref[...],
                              preferred_element_type=jnp.float32)
                      + ba2_ref[...])


def kernel(x, edge_index, batch, W0, b0, Wg, att_src, att_dst, bg,
           Wa1, ba1, Wa2, ba2):
    loops = jnp.arange(_N, dtype=edge_index.dtype)
    src = jnp.concatenate([edge_index[0], loops])
    dst = jnp.concatenate([edge_index[1], loops])
    order = jnp.argsort(dst)
    srcs = src[order].astype(jnp.int32)
    dsts = dst[order].astype(jnp.int32)
    srcs_pad = jnp.concatenate(
        [srcs, jnp.zeros((_EPAD - _ETOT,), jnp.int32)])
    rp_pad = jnp.searchsorted(
        dsts, jnp.arange(_NP + 16, dtype=jnp.int32), side="left"
    ).astype(jnp.float32)
    x_pad = jnp.pad(x, ((0, _NP - _N), (0, 0)))
    batch_pad = jnp.pad(batch.astype(jnp.int32), (0, _NP - _N),
                        constant_values=_B).reshape(1, _NP)
    b0_2d = b0.reshape(1, _HID)

    bn = 1024
    h = pl.pallas_call(
        _in_proj_body,
        grid=(_NP // bn,),
        in_specs=[
            pl.BlockSpec((bn, _DIN), lambda i: (i, 0)),
            pl.BlockSpec((_DIN, _HID), lambda i: (0, 0)),
            pl.BlockSpec((1, _HID), lambda i: (0, 0)),
        ],
        out_specs=pl.BlockSpec((bn, _HID), lambda i: (i, 0)),
        out_shape=jax.ShapeDtypeStruct((_NP, _HID), jnp.float32),
    )(x_pad, W0, b0_2d)

    bx = 512
    for i in range(_L):
        asp = jnp.pad(att_src[i], ((0, 8 - _HEADS), (0, 0)))
        adp = jnp.pad(att_dst[i], ((0, 8 - _HEADS), (0, 0)))
        xp, tbl = pl.pallas_call(
            _xp_body,
            grid=(_NP // bx,),
            in_specs=[
                pl.BlockSpec((bx, _HID), lambda j: (j, 0)),
                pl.BlockSpec((_HID, _HEADS * _HID), lambda j: (0, 0)),
                pl.BlockSpec((8, _HID), lambda j: (0, 0)),
                pl.BlockSpec((8, _HID), lambda j: (0, 0)),
            ],
            out_specs=[
                pl.BlockSpec((bx, _HEADS * _HID), lambda j: (j, 0)),
                pl.BlockSpec((8, bx), lambda j: (0, j)),
            ],
            out_shape=[
                jax.ShapeDtypeStruct((_NP, _HEADS * _HID), jnp.float32),
                jax.ShapeDtypeStruct((8, _NP), jnp.float32),
            ],
        )(h, Wg[i], asp, adp)

        out_sum = _gat_aggregate(srcs_pad, rp_pad, tbl.reshape(-1), xp)

        h = pl.pallas_call(
            _upd_body,
            grid=(_NP // bx,),
            in_specs=[
                pl.BlockSpec((bx, _HEADS * _HID), lambda j: (j, 0)),
                pl.BlockSpec((bx, _HID), lambda j: (j, 0)),
                pl.BlockSpec((1, _HID), lambda j: (0, 0)),
            ],
            out_specs=pl.BlockSpec((bx, _HID), lambda j: (j, 0)),
            out_shape=jax.ShapeDtypeStruct((_NP, _HID), jnp.float32),
        )(out_sum, h, bg[i].reshape(1, _HID))

    bp = 2048
    out = pl.pallas_call(
        _pool_body,
        grid=(_NP // bp,),
        in_specs=[
            pl.BlockSpec((bp, _HID), lambda j: (j, 0)),
            pl.BlockSpec((1, bp), lambda j: (0, j)),
            pl.BlockSpec((_HID, 2 * _HID), lambda j: (0, 0)),
            pl.BlockSpec((1, 2 * _HID), lambda j: (0, 0)),
            pl.BlockSpec((2 * _HID, _HID), lambda j: (0, 0)),
            pl.BlockSpec((1, _HID), lambda j: (0, 0)),
        ],
        out_specs=pl.BlockSpec((_B, _HID), lambda j: (0, 0)),
        out_shape=jax.ShapeDtypeStruct((_B, _HID), jnp.float32),
        scratch_shapes=[
            pltpu.VMEM((_B, _HID), jnp.float32),
            pltpu.VMEM((_B, 1), jnp.float32),
        ],
    )(h, batch_pad, Wa1, ba1.reshape(1, 2 * _HID), Wa2,
      ba2.reshape(1, _HID))
    return out


# speedup vs baseline: 13.1438x; 2.4474x over previous
import functools

import jax
import jax.numpy as jnp
from jax import lax
from jax.experimental import pallas as pl
from jax.experimental.pallas import tpu as pltpu
from jax.experimental.pallas import tpu_sc as plsc

_N = 10000
_E = 320000
_DIN = 128
_HID = 256
_HEADS = 4
_B = 64
_L = 3

_NP = 10240
_NW = 32
_NPT = _NP // _NW
_ETOT = _E + _N
_EPAD = 330160
_NEG = -3.0e38


def _full16(val):
    return jnp.full((16,), val, jnp.int32)


def _exti(ref, pos):
    blk = pl.multiple_of(jnp.bitwise_and(pos, jnp.int32(-16)), 16)
    v = ref[pl.ds(blk, 16)]
    lane = lax.iota(jnp.int32, 16) + blk
    s = jnp.sum(jnp.where(lane == pos, v, jnp.float32(0.0)))
    return s.astype(jnp.int32)


def _gat_sc_body(srcs_h, rp_h, tbl_h, xp_h, out_h,
                 tbl_v, rp_v, idxa_v, idxb_v, rows_a, rows_b, acc_v, ebuf_v,
                 sem_a, sem_b):
    c = lax.axis_index("c")
    s = lax.axis_index("s")
    wid = s * 2 + c
    n0 = pl.multiple_of(wid * _NPT, _NPT)
    pltpu.sync_copy(tbl_h, tbl_v)
    pltpu.sync_copy(rp_h.at[pl.ds(n0, _NPT + 16)], rp_v)
    lanes = lax.iota(jnp.int32, 16)

    def node_body(nl, _):
        n = n0 + nl
        e0 = _exti(rp_v, nl)
        e1 = _exti(rp_v, nl + 1)
        a0 = pl.multiple_of(jnp.bitwise_and(e0, jnp.int32(-16)), 16)
        nblk = (e1 - a0 + 15) >> 4
        nvec = _full16(n)
        adv = [plsc.load_gather(tbl_v, [_full16((4 + h) * _NP) + nvec])
               for h in range(_HEADS)]

        def leaky_alpha(sidx, h):
            asv = plsc.load_gather(tbl_v, [_full16(h * _NP) + sidx])
            al = asv + adv[h]
            return jnp.where(al >= 0.0, al, al * jnp.float32(0.2))

        nch = (nblk + 3) >> 2

        def ch_a(cc, ms):
            cb = pl.multiple_of(a0 + cc * 64, 16)
            pltpu.sync_copy(srcs_h.at[pl.ds(cb, 64)], idxa_v)
            new = list(ms)
            for j in range(4):
                eb = cb + j * 16
                sidx = idxa_v[pl.ds(j * 16, 16)]
                valid = (lanes + eb >= e0) & (lanes + eb < e1)
                for h in range(_HEADS):
                    al = leaky_alpha(sidx, h)
                    new[h] = jnp.maximum(new[h], jnp.where(valid, al, _NEG))
            return tuple(new)

        ms = lax.fori_loop(0, nch, ch_a,
                           tuple(jnp.full((16,), _NEG, jnp.float32)
                                 for _ in range(_HEADS)))
        amax = [jnp.max(ms[h]) for h in range(_HEADS)]

        for t in range(64):
            acc_v[pl.ds(t * 16, 16)] = jnp.zeros((16,), jnp.float32)

        def block_evs(eb, sidx):
            valid = (lanes + eb >= e0) & (lanes + eb < e1)
            evs = []
            for h in range(_HEADS):
                al = leaky_alpha(sidx, h)
                ev = jnp.exp(al - amax[h])
                evs.append(jnp.where(valid, ev, jnp.float32(0.0)))
            return evs

        def process_block(rows_ref, evs):
            for h in range(_HEADS):
                ebuf_v[pl.ds(h * 16, 16)] = evs[h]
            for h in range(_HEADS):
                ebc = [plsc.load_gather(ebuf_v, [_full16(h * 16 + r)])
                       for r in range(16)]

                def tt_body(tt, _, h=h, ebc=ebc):
                    off = pl.multiple_of((h * 16 + tt) * 16, 16)
                    col = _full16((h * 16 + tt) * 16) + lanes
                    a = acc_v[pl.ds(off, 16)]
                    for r in range(16):
                        rch = plsc.load_gather(rows_ref, [_full16(r), col])
                        a = a + rch * ebc[r]
                    acc_v[pl.ds(off, 16)] = a
                    return 0

                lax.fori_loop(0, 16, tt_body, 0)

        npair = (nblk + 1) >> 1

        def pair_body(k, dsum):
            b0 = k * 2
            cb = pl.multiple_of(a0 + b0 * 16, 16)
            pltpu.sync_copy(srcs_h.at[pl.ds(cb, 32)], idxb_v)
            sidx0 = idxb_v[pl.ds(0, 16)]
            sidx1 = idxb_v[pl.ds(16, 16)]
            cp0 = pltpu.async_copy(xp_h.at[sidx0], rows_a, sem_a)
            has1 = (b0 + 1) < nblk

            @pl.when(has1)
            def _():
                pltpu.async_copy(xp_h.at[sidx1], rows_b, sem_b)

            evs0 = block_evs(cb, sidx0)
            evs1 = block_evs(cb + 16, sidx1)
            cp0.wait()
            process_block(rows_a, evs0)

            @pl.when(has1)
            def _():
                pltpu.make_async_copy(xp_h.at[sidx1], rows_b, sem_b).wait()
                process_block(rows_b, evs1)

            return tuple(dsum[h] + evs0[h] + evs1[h] for h in range(_HEADS))

        dsum = lax.fori_loop(0, npair, pair_body,
                             tuple(jnp.zeros((16,), jnp.float32)
                                   for _ in range(_HEADS)))
        ones16 = jnp.ones((16,), jnp.float32)
        for h in range(_HEADS):
            dvec = jnp.full((16,), jnp.sum(dsum[h]), jnp.float32)
            inv = ones16 / (dvec + jnp.float32(1e-16))
            for t in range(16):
                off = h * 256 + t * 16
                acc_v[pl.ds(off, 16)] = acc_v[pl.ds(off, 16)] * inv
        pltpu.sync_copy(acc_v, out_h.at[n])
        return 0

    lax.fori_loop(0, _NPT, node_body, 0)


def _gat_aggregate(srcs_pad, rp_pad, tbl, xp):
    mesh = plsc.VectorSubcoreMesh(core_axis_name="c", subcore_axis_name="s",
                                  num_cores=2, num_subcores=16)
    return pl.kernel(
        _gat_sc_body,
        out_type=jax.ShapeDtypeStruct((_NP, _HEADS * _HID), jnp.float32),
        mesh=mesh,
        scratch_types=[
            pltpu.VMEM((8 * _NP,), jnp.float32),
            pltpu.VMEM((_NPT + 16,), jnp.float32),
            pltpu.VMEM((64,), jnp.int32),
            pltpu.VMEM((32,), jnp.int32),
            pltpu.VMEM((16, _HEADS * _HID), jnp.float32),
            pltpu.VMEM((16, _HEADS * _HID), jnp.float32),
            pltpu.VMEM((_HEADS * _HID,), jnp.float32),
            pltpu.VMEM((_HEADS * 16,), jnp.float32),
            pltpu.SemaphoreType.DMA,
            pltpu.SemaphoreType.DMA,
        ],
        compiler_params=pltpu.CompilerParams(needs_layout_passes=False),
    )(srcs_pad, rp_pad, tbl, xp)


def _in_proj_body(x_ref, w_ref, b_ref, o_ref):
    o_ref[...] = jnp.maximum(
        jnp.dot(x_ref[...], w_ref[...], preferred_element_type=jnp.float32)
        + b_ref[...], 0.0)


def _xp_body(h_ref, w_ref, as_ref, ad_ref, xp_ref, tb_ref):
    xp = jnp.dot(h_ref[...], w_ref[...], preferred_element_type=jnp.float32)
    xp_ref[...] = xp
    rows = []
    for h in range(_HEADS):
        xh = xp[:, h * _HID:(h + 1) * _HID]
        rows.append(jnp.sum(xh * as_ref[h, :][None, :], axis=1)[None, :])
    for h in range(_HEADS):
        xh = xp[:, h * _HID:(h + 1) * _HID]
        rows.append(jnp.sum(xh * ad_ref[h, :][None, :], axis=1)[None, :])
    tb_ref[...] = jnp.concatenate(rows, axis=0)


def _upd_body(os_ref, h_ref, bg_ref, o_ref):
    i = pl.program_id(0)
    sblk = os_ref[...]
    hm = (sblk[:, 0:256] + sblk[:, 256:512]
          + sblk[:, 512:768] + sblk[:, 768:1024]) * 0.25
    hn = jnp.maximum(hm + bg_ref[...], 0.0)
    rows = i * sblk.shape[0] + lax.broadcasted_iota(
        jnp.int32, (sblk.shape[0], _HID), 0)
    o_ref[...] = h_ref[...] + jnp.where(rows < _N, hn, 0.0)


def _pool_body(h_ref, b_ref, wa1_ref, ba1_ref, wa2_ref, ba2_ref, o_ref,
               acc, cnt):
    j = pl.program_id(0)

    @pl.when(j == 0)
    def _():
        acc[...] = jnp.zeros_like(acc)
        cnt[...] = jnp.zeros_like(cnt)

    bn = h_ref.shape[0]
    P = (b_ref[...] == lax.broadcasted_iota(jnp.int32, (_B, bn), 0)
         ).astype(jnp.float32)
    acc[...] += jnp.dot(P, h_ref[...], preferred_element_type=jnp.float32)
    cnt[...] += jnp.sum(P, axis=1, keepdims=True)

    @pl.when(j == pl.num_programs(0) - 1)
    def _():
        pooled = acc[...] / jnp.maximum(cnt[...][:, 0:1], 1.0)
        t = jnp.maximum(
            jnp.dot(pooled, wa1_ref[...], preferred_element_type=jnp.float32)
            + ba1_ref[...], 0.0)
        o_ref[...] = (jnp.dot(t, wa2_ref[...],
                              preferred_element_type=jnp.float32)
                      + ba2_ref[...])


def kernel(x, edge_index, batch, W0, b0, Wg, att_src, att_dst, bg,
           Wa1, ba1, Wa2, ba2):
    loops = jnp.arange(_N, dtype=edge_index.dtype)
    src = jnp.concatenate([edge_index[0], loops])
    dst = jnp.concatenate([edge_index[1], loops])
    order = jnp.argsort(dst)
    srcs = src[order].astype(jnp.int32)
    dsts = dst[order].astype(jnp.int32)
    srcs_pad = jnp.concatenate(
        [srcs, jnp.zeros((_EPAD - _ETOT,), jnp.int32)])
    rp_pad = jnp.searchsorted(
        dsts, jnp.arange(_NP + 16, dtype=jnp.int32), side="left"
    ).astype(jnp.float32)
    x_pad = jnp.pad(x, ((0, _NP - _N), (0, 0)))
    batch_pad = jnp.pad(batch.astype(jnp.int32), (0, _NP - _N),
                        constant_values=_B).reshape(1, _NP)
    b0_2d = b0.reshape(1, _HID)

    bn = 1024
    h = pl.pallas_call(
        _in_proj_body,
        grid=(_NP // bn,),
        in_specs=[
            pl.BlockSpec((bn, _DIN), lambda i: (i, 0)),
            pl.BlockSpec((_DIN, _HID), lambda i: (0, 0)),
            pl.BlockSpec((1, _HID), lambda i: (0, 0)),
        ],
        out_specs=pl.BlockSpec((bn, _HID), lambda i: (i, 0)),
        out_shape=jax.ShapeDtypeStruct((_NP, _HID), jnp.float32),
    )(x_pad, W0, b0_2d)

    bx = 512
    for i in range(_L):
        asp = jnp.pad(att_src[i], ((0, 8 - _HEADS), (0, 0)))
        adp = jnp.pad(att_dst[i], ((0, 8 - _HEADS), (0, 0)))
        xp, tbl = pl.pallas_call(
            _xp_body,
            grid=(_NP // bx,),
            in_specs=[
                pl.BlockSpec((bx, _HID), lambda j: (j, 0)),
                pl.BlockSpec((_HID, _HEADS * _HID), lambda j: (0, 0)),
                pl.BlockSpec((8, _HID), lambda j: (0, 0)),
                pl.BlockSpec((8, _HID), lambda j: (0, 0)),
            ],
            out_specs=[
                pl.BlockSpec((bx, _HEADS * _HID), lambda j: (j, 0)),
                pl.BlockSpec((8, bx), lambda j: (0, j)),
            ],
            out_shape=[
                jax.ShapeDtypeStruct((_NP, _HEADS * _HID), jnp.float32),
                jax.ShapeDtypeStruct((8, _NP), jnp.float32),
            ],
        )(h, Wg[i], asp, adp)

        out_sum = _gat_aggregate(srcs_pad, rp_pad, tbl.reshape(-1), xp)

        h = pl.pallas_call(
            _upd_body,
            grid=(_NP // bx,),
            in_specs=[
                pl.BlockSpec((bx, _HEADS * _HID), lambda j: (j, 0)),
                pl.BlockSpec((bx, _HID), lambda j: (j, 0)),
                pl.BlockSpec((1, _HID), lambda j: (0, 0)),
            ],
            out_specs=pl.BlockSpec((bx, _HID), lambda j: (j, 0)),
            out_shape=jax.ShapeDtypeStruct((_NP, _HID), jnp.float32),
        )(out_sum, h, bg[i].reshape(1, _HID))

    bp = 2048
    out = pl.pallas_call(
        _pool_body,
        grid=(_NP // bp,),
        in_specs=[
            pl.BlockSpec((bp, _HID), lambda j: (j, 0)),
            pl.BlockSpec((1, bp), lambda j: (0, j)),
            pl.BlockSpec((_HID, 2 * _HID), lambda j: (0, 0)),
            pl.BlockSpec((1, 2 * _HID), lambda j: (0, 0)),
            pl.BlockSpec((2 * _HID, _HID), lambda j: (0, 0)),
            pl.BlockSpec((1, _HID), lambda j: (0, 0)),
        ],
        out_specs=pl.BlockSpec((_B, _HID), lambda j: (0, 0)),
        out_shape=jax.ShapeDtypeStruct((_B, _HID), jnp.float32),
        scratch_shapes=[
            pltpu.VMEM((_B, _HID), jnp.float32),
            pltpu.VMEM((_B, 1), jnp.float32),
        ],
    )(h, batch_pad, Wa1, ba1.reshape(1, 2 * _HID), Wa2,
      ba2.reshape(1, _HID))
    return out


# speedup vs baseline: 13.7904x; 1.0492x over previous
import functools

import jax
import jax.numpy as jnp
from jax import lax
from jax.experimental import pallas as pl
from jax.experimental.pallas import tpu as pltpu
from jax.experimental.pallas import tpu_sc as plsc

_N = 10000
_E = 320000
_DIN = 128
_HID = 256
_HEADS = 4
_B = 64
_L = 3

_NP = 10240
_NW = 32
_NPT = _NP // _NW
_ETOT = _E + _N
_EPAD = 330160
_NEG = -3.0e38


def _full16(val):
    return jnp.full((16,), val, jnp.int32)


def _exti(ref, pos):
    blk = pl.multiple_of(jnp.bitwise_and(pos, jnp.int32(-16)), 16)
    v = ref[pl.ds(blk, 16)]
    lane = lax.iota(jnp.int32, 16) + blk
    s = jnp.sum(jnp.where(lane == pos, v, jnp.float32(0.0)))
    return s.astype(jnp.int32)


_NL = _NPT + 16


def _gat_sc_body(srcs_h, rp_h, tbl_h, xp_h, out_h,
                 asrc_v, adst_v, rp_v, idxa_v, idxb_v,
                 rows0, rows1, rows2, rows3, acc_v, ebuf_v,
                 sem0, sem1, sem2, sem3):
    c = lax.axis_index("c")
    s = lax.axis_index("s")
    wid = s * 2 + c
    n0 = pl.multiple_of(wid * _NPT, _NPT)
    pltpu.sync_copy(tbl_h.at[pl.ds(0, 4 * _NP)], asrc_v)
    for h in range(_HEADS):
        pltpu.sync_copy(tbl_h.at[pl.ds((4 + h) * _NP + n0, _NL)],
                        adst_v.at[pl.ds(h * _NL, _NL)])
    pltpu.sync_copy(rp_h.at[pl.ds(n0, _NL)], rp_v)
    lanes = lax.iota(jnp.int32, 16)

    def node_body(nl, _):
        n = n0 + nl
        e0 = _exti(rp_v, nl)
        e1 = _exti(rp_v, nl + 1)
        a0 = pl.multiple_of(jnp.bitwise_and(e0, jnp.int32(-16)), 16)
        nblk = (e1 - a0 + 15) >> 4
        nlv = _full16(nl)
        adv = [plsc.load_gather(adst_v, [_full16(h * _NL) + nlv])
               for h in range(_HEADS)]

        def leaky_alpha(sidx, h):
            asv = plsc.load_gather(asrc_v, [_full16(h * _NP) + sidx])
            al = asv + adv[h]
            return jnp.where(al >= 0.0, al, al * jnp.float32(0.2))

        nch = (nblk + 3) >> 2

        def ch_a(cc, ms):
            cb = pl.multiple_of(a0 + cc * 64, 16)
            pltpu.sync_copy(srcs_h.at[pl.ds(cb, 64)], idxa_v)
            new = list(ms)
            for j in range(4):
                eb = cb + j * 16
                sidx = idxa_v[pl.ds(j * 16, 16)]
                valid = (lanes + eb >= e0) & (lanes + eb < e1)
                for h in range(_HEADS):
                    al = leaky_alpha(sidx, h)
                    new[h] = jnp.maximum(new[h], jnp.where(valid, al, _NEG))
            return tuple(new)

        ms = lax.fori_loop(0, nch, ch_a,
                           tuple(jnp.full((16,), _NEG, jnp.float32)
                                 for _ in range(_HEADS)))
        amax = [jnp.max(ms[h]) for h in range(_HEADS)]

        for t in range(64):
            acc_v[pl.ds(t * 16, 16)] = jnp.zeros((16,), jnp.float32)

        def block_evs(eb, sidx):
            valid = (lanes + eb >= e0) & (lanes + eb < e1)
            evs = []
            for h in range(_HEADS):
                al = leaky_alpha(sidx, h)
                ev = jnp.exp(al - amax[h])
                evs.append(jnp.where(valid, ev, jnp.float32(0.0)))
            return evs

        def process_block(rows_ref, evs):
            for h in range(_HEADS):
                ebuf_v[pl.ds(h * 16, 16)] = evs[h]
            for h in range(_HEADS):
                ebc = [plsc.load_gather(ebuf_v, [_full16(h * 16 + r)])
                       for r in range(16)]

                def tt_body(tt, _, h=h, ebc=ebc):
                    off = pl.multiple_of((h * 16 + tt) * 16, 16)
                    col = _full16((h * 16 + tt) * 16) + lanes
                    a = acc_v[pl.ds(off, 16)]
                    for r in range(16):
                        rch = plsc.load_gather(rows_ref, [_full16(r), col])
                        a = a + rch * ebc[r]
                    acc_v[pl.ds(off, 16)] = a
                    return 0

                lax.fori_loop(0, 16, tt_body, 0)

        nquad = (nblk + 3) >> 2
        rbufs = [(rows0, sem0), (rows1, sem1), (rows2, sem2), (rows3, sem3)]

        def quad_body(k, dsum):
            b0 = k * 4
            cb = pl.multiple_of(a0 + b0 * 16, 16)
            pltpu.sync_copy(srcs_h.at[pl.ds(cb, 64)], idxb_v)
            sidx = [idxb_v[pl.ds(j * 16, 16)] for j in range(4)]
            cp0 = pltpu.async_copy(xp_h.at[sidx[0]], rows0, sem0)
            for j in range(1, 4):
                @pl.when(b0 + j < nblk)
                def _(j=j):
                    rb, sm = rbufs[j]
                    pltpu.async_copy(xp_h.at[sidx[j]], rb, sm)

            evs = [block_evs(cb + 16 * j, sidx[j]) for j in range(4)]
            cp0.wait()
            process_block(rows0, evs[0])
            for j in range(1, 4):
                @pl.when(b0 + j < nblk)
                def _(j=j):
                    rb, sm = rbufs[j]
                    pltpu.make_async_copy(xp_h.at[sidx[j]], rb, sm).wait()
                    process_block(rb, evs[j])

            return tuple(dsum[h] + evs[0][h] + evs[1][h] + evs[2][h]
                         + evs[3][h] for h in range(_HEADS))

        dsum = lax.fori_loop(0, nquad, quad_body,
                             tuple(jnp.zeros((16,), jnp.float32)
                                   for _ in range(_HEADS)))
        ones16 = jnp.ones((16,), jnp.float32)
        for h in range(_HEADS):
            dvec = jnp.full((16,), jnp.sum(dsum[h]), jnp.float32)
            inv = ones16 / (dvec + jnp.float32(1e-16))
            for t in range(16):
                off = h * 256 + t * 16
                acc_v[pl.ds(off, 16)] = acc_v[pl.ds(off, 16)] * inv
        pltpu.sync_copy(acc_v, out_h.at[n])
        return 0

    lax.fori_loop(0, _NPT, node_body, 0)


def _gat_aggregate(srcs_pad, rp_pad, tbl, xp):
    mesh = plsc.VectorSubcoreMesh(core_axis_name="c", subcore_axis_name="s",
                                  num_cores=2, num_subcores=16)
    return pl.kernel(
        _gat_sc_body,
        out_type=jax.ShapeDtypeStruct((_NP, _HEADS * _HID), jnp.float32),
        mesh=mesh,
        scratch_types=[
            pltpu.VMEM((4 * _NP,), jnp.float32),
            pltpu.VMEM((_HEADS * _NL,), jnp.float32),
            pltpu.VMEM((_NL,), jnp.float32),
            pltpu.VMEM((64,), jnp.int32),
            pltpu.VMEM((64,), jnp.int32),
            pltpu.VMEM((16, _HEADS * _HID), jnp.float32),
            pltpu.VMEM((16, _HEADS * _HID), jnp.float32),
            pltpu.VMEM((16, _HEADS * _HID), jnp.float32),
            pltpu.VMEM((16, _HEADS * _HID), jnp.float32),
            pltpu.VMEM((_HEADS * _HID,), jnp.float32),
            pltpu.VMEM((_HEADS * 16,), jnp.float32),
            pltpu.SemaphoreType.DMA,
            pltpu.SemaphoreType.DMA,
            pltpu.SemaphoreType.DMA,
            pltpu.SemaphoreType.DMA,
        ],
        compiler_params=pltpu.CompilerParams(needs_layout_passes=False),
    )(srcs_pad, rp_pad, tbl, xp)


def _in_proj_body(x_ref, w_ref, b_ref, o_ref):
    o_ref[...] = jnp.maximum(
        jnp.dot(x_ref[...], w_ref[...], preferred_element_type=jnp.float32)
        + b_ref[...], 0.0)


def _xp_body(h_ref, w_ref, as_ref, ad_ref, xp_ref, tb_ref):
    xp = jnp.dot(h_ref[...], w_ref[...], preferred_element_type=jnp.float32)
    xp_ref[...] = xp
    rows = []
    for h in range(_HEADS):
        xh = xp[:, h * _HID:(h + 1) * _HID]
        rows.append(jnp.sum(xh * as_ref[h, :][None, :], axis=1)[None, :])
    for h in range(_HEADS):
        xh = xp[:, h * _HID:(h + 1) * _HID]
        rows.append(jnp.sum(xh * ad_ref[h, :][None, :], axis=1)[None, :])
    tb_ref[...] = jnp.concatenate(rows, axis=0)


def _upd_body(os_ref, h_ref, bg_ref, o_ref):
    i = pl.program_id(0)
    sblk = os_ref[...]
    hm = (sblk[:, 0:256] + sblk[:, 256:512]
          + sblk[:, 512:768] + sblk[:, 768:1024]) * 0.25
    hn = jnp.maximum(hm + bg_ref[...], 0.0)
    rows = i * sblk.shape[0] + lax.broadcasted_iota(
        jnp.int32, (sblk.shape[0], _HID), 0)
    o_ref[...] = h_ref[...] + jnp.where(rows < _N, hn, 0.0)


def _pool_body(h_ref, b_ref, wa1_ref, ba1_ref, wa2_ref, ba2_ref, o_ref,
               acc, cnt):
    j = pl.program_id(0)

    @pl.when(j == 0)
    def _():
        acc[...] = jnp.zeros_like(acc)
        cnt[...] = jnp.zeros_like(cnt)

    bn = h_ref.shape[0]
    P = (b_ref[...] == lax.broadcasted_iota(jnp.int32, (_B, bn), 0)
         ).astype(jnp.float32)
    acc[...] += jnp.dot(P, h_ref[...], preferred_element_type=jnp.float32)
    cnt[...] += jnp.sum(P, axis=1, keepdims=True)

    @pl.when(j == pl.num_programs(0) - 1)
    def _():
        pooled = acc[...] / jnp.maximum(cnt[...][:, 0:1], 1.0)
        t = jnp.maximum(
            jnp.dot(pooled, wa1_ref[...], preferred_element_type=jnp.float32)
            + ba1_ref[...], 0.0)
        o_ref[...] = (jnp.dot(t, wa2_ref[...],
                              preferred_element_type=jnp.float32)
                      + ba2_ref[...])


def kernel(x, edge_index, batch, W0, b0, Wg, att_src, att_dst, bg,
           Wa1, ba1, Wa2, ba2):
    loops = jnp.arange(_N, dtype=edge_index.dtype)
    src = jnp.concatenate([edge_index[0], loops])
    dst = jnp.concatenate([edge_index[1], loops])
    order = jnp.argsort(dst)
    srcs = src[order].astype(jnp.int32)
    dsts = dst[order].astype(jnp.int32)
    srcs_pad = jnp.concatenate(
        [srcs, jnp.zeros((_EPAD - _ETOT,), jnp.int32)])
    rp_pad = jnp.searchsorted(
        dsts, jnp.arange(_NP + 16, dtype=jnp.int32), side="left"
    ).astype(jnp.float32)
    x_pad = jnp.pad(x, ((0, _NP - _N), (0, 0)))
    batch_pad = jnp.pad(batch.astype(jnp.int32), (0, _NP - _N),
                        constant_values=_B).reshape(1, _NP)
    b0_2d = b0.reshape(1, _HID)

    bn = 1024
    h = pl.pallas_call(
        _in_proj_body,
        grid=(_NP // bn,),
        in_specs=[
            pl.BlockSpec((bn, _DIN), lambda i: (i, 0)),
            pl.BlockSpec((_DIN, _HID), lambda i: (0, 0)),
            pl.BlockSpec((1, _HID), lambda i: (0, 0)),
        ],
        out_specs=pl.BlockSpec((bn, _HID), lambda i: (i, 0)),
        out_shape=jax.ShapeDtypeStruct((_NP, _HID), jnp.float32),
    )(x_pad, W0, b0_2d)

    bx = 512
    for i in range(_L):
        asp = jnp.pad(att_src[i], ((0, 8 - _HEADS), (0, 0)))
        adp = jnp.pad(att_dst[i], ((0, 8 - _HEADS), (0, 0)))
        xp, tbl = pl.pallas_call(
            _xp_body,
            grid=(_NP // bx,),
            in_specs=[
                pl.BlockSpec((bx, _HID), lambda j: (j, 0)),
                pl.BlockSpec((_HID, _HEADS * _HID), lambda j: (0, 0)),
                pl.BlockSpec((8, _HID), lambda j: (0, 0)),
                pl.BlockSpec((8, _HID), lambda j: (0, 0)),
            ],
            out_specs=[
                pl.BlockSpec((bx, _HEADS * _HID), lambda j: (j, 0)),
                pl.BlockSpec((8, bx), lambda j: (0, j)),
            ],
            out_shape=[
                jax.ShapeDtypeStruct((_NP, _HEADS * _HID), jnp.float32),
                jax.ShapeDtypeStruct((8, _NP), jnp.float32),
            ],
        )(h, Wg[i], asp, adp)

        out_sum = _gat_aggregate(srcs_pad, rp_pad, tbl.reshape(-1), xp)

        h = pl.pallas_call(
            _upd_body,
            grid=(_NP // bx,),
            in_specs=[
                pl.BlockSpec((bx, _HEADS * _HID), lambda j: (j, 0)),
                pl.BlockSpec((bx, _HID), lambda j: (j, 0)),
                pl.BlockSpec((1, _HID), lambda j: (0, 0)),
            ],
            out_specs=pl.BlockSpec((bx, _HID), lambda j: (j, 0)),
            out_shape=jax.ShapeDtypeStruct((_NP, _HID), jnp.float32),
        )(out_sum, h, bg[i].reshape(1, _HID))

    bp = 2048
    out = pl.pallas_call(
        _pool_body,
        grid=(_NP // bp,),
        in_specs=[
            pl.BlockSpec((bp, _HID), lambda j: (j, 0)),
            pl.BlockSpec((1, bp), lambda j: (0, j)),
            pl.BlockSpec((_HID, 2 * _HID), lambda j: (0, 0)),
            pl.BlockSpec((1, 2 * _HID), lambda j: (0, 0)),
            pl.BlockSpec((2 * _HID, _HID), lambda j: (0, 0)),
            pl.BlockSpec((1, _HID), lambda j: (0, 0)),
        ],
        out_specs=pl.BlockSpec((_B, _HID), lambda j: (0, 0)),
        out_shape=jax.ShapeDtypeStruct((_B, _HID), jnp.float32),
        scratch_shapes=[
            pltpu.VMEM((_B, _HID), jnp.float32),
            pltpu.VMEM((_B, 1), jnp.float32),
        ],
    )(h, batch_pad, Wa1, ba1.reshape(1, 2 * _HID), Wa2,
      ba2.reshape(1, _HID))
    return out
